# baseline (speedup 1.0000x reference)
import jax
import jax.numpy as jnp
from jax import lax
from jax.experimental import pallas as pl

_K = 8


def _chamfer_body(a1_ref, a2_ref, out_ref):
    b = pl.program_id(0)
    nbatch = pl.num_programs(0)
    f32 = jnp.float32
    a1 = a1_ref[0]
    a2 = a2_ref[0]
    n = a1.shape[0]
    m = a2.shape[0]
    n1 = jnp.sum(a1 * a1, axis=1, keepdims=True)
    n2 = jnp.sum(a2 * a2, axis=1, keepdims=True)
    aug1 = jnp.concatenate(
        [-2.0 * a1, n1, jnp.ones((n, 1), f32), jnp.zeros((n, _K - 5), f32)],
        axis=1)
    aug2 = jnp.concatenate(
        [a2, jnp.ones((m, 1), f32), n2, jnp.zeros((m, _K - 5), f32)],
        axis=1)
    d = lax.dot_general(aug1, aug2, (((1,), (1,)), ((), ())),
                        preferred_element_type=f32)
    rowmin = jnp.maximum(jnp.min(d, axis=1, keepdims=True), 0.0)
    colmin = jnp.maximum(jnp.min(d, axis=0, keepdims=True), 0.0)
    s = (jnp.sum(rowmin, axis=(0, 1), keepdims=True) / (nbatch * n)
         + jnp.sum(colmin, axis=(0, 1), keepdims=True) / (nbatch * m))

    @pl.when(b == 0)
    def _():
        out_ref[...] = s

    @pl.when(b != 0)
    def _():
        out_ref[...] = out_ref[...] + s


def kernel(array1, array2):
    B, N, _ = array1.shape
    M = array2.shape[1]
    out = pl.pallas_call(
        _chamfer_body,
        grid=(B,),
        in_specs=[
            pl.BlockSpec((1, N, 3), lambda b: (b, 0, 0)),
            pl.BlockSpec((1, M, 3), lambda b: (b, 0, 0)),
        ],
        out_specs=pl.BlockSpec((1, 1), lambda b: (0, 0)),
        out_shape=jax.ShapeDtypeStruct((1, 1), jnp.float32),
    )(array1, array2)
    return out[0, 0]


# speedup vs baseline: 1.0557x; 1.0557x over previous
import jax
import jax.numpy as jnp
from jax import lax
from jax.experimental import pallas as pl

_K = 8


def _chamfer_body(a1_ref, a2t_ref, out_ref):
    f32 = jnp.float32
    nbatch, n, _ = a1_ref.shape
    m = a2t_ref.shape[2]

    def step(b, acc):
        a1 = a1_ref[b]
        a2t = a2t_ref[b]
        n1 = jnp.sum(a1 * a1, axis=1, keepdims=True)
        n2 = jnp.sum(a2t * a2t, axis=0, keepdims=True)
        aug1 = jnp.concatenate(
            [-2.0 * a1, n1, jnp.ones((n, 1), f32),
             jnp.zeros((n, _K - 5), f32)], axis=1)
        aug2 = jnp.concatenate(
            [a2t, jnp.ones((1, m), f32), n2,
             jnp.zeros((_K - 5, m), f32)], axis=0)
        d = jnp.dot(aug1, aug2, preferred_element_type=f32)
        rowmin = jnp.maximum(jnp.min(d, axis=1, keepdims=True), 0.0)
        colmin = jnp.maximum(jnp.min(d, axis=0, keepdims=True), 0.0)
        return (acc
                + jnp.sum(rowmin, axis=(0, 1), keepdims=True) / (nbatch * n)
                + jnp.sum(colmin, axis=(0, 1), keepdims=True) / (nbatch * m))

    out_ref[...] = lax.fori_loop(0, nbatch, step, jnp.zeros((1, 1), f32))


def kernel(array1, array2):
    B, N, _ = array1.shape
    M = array2.shape[1]
    a2t = jnp.swapaxes(array2, 1, 2)
    out = pl.pallas_call(
        _chamfer_body,
        in_specs=[
            pl.BlockSpec((B, N, 3), lambda: (0, 0, 0)),
            pl.BlockSpec((B, 3, M), lambda: (0, 0, 0)),
        ],
        out_specs=pl.BlockSpec((1, 1), lambda: (0, 0)),
        out_shape=jax.ShapeDtypeStruct((1, 1), jnp.float32),
    )(array1, a2t)
    return out[0, 0]


# speedup vs baseline: 1.1095x; 1.0510x over previous
import jax
import jax.numpy as jnp
from jax.experimental import pallas as pl

_K = 8
_MC = 512


def _chamfer_body(a1_ref, a2t_ref, out_ref):
    b = pl.program_id(0)
    nbatch = pl.num_programs(0)
    f32 = jnp.float32
    a1 = a1_ref[0]
    a2t = a2t_ref[0]
    n = a1.shape[0]
    m = a2t.shape[1]
    n1 = jnp.sum(a1 * a1, axis=1, keepdims=True)
    n2 = jnp.sum(a2t * a2t, axis=0, keepdims=True)
    aug1 = jnp.concatenate(
        [-2.0 * a1, n1, jnp.ones((n, 1), f32), jnp.zeros((n, _K - 5), f32)],
        axis=1)
    aug2 = jnp.concatenate(
        [a2t, jnp.ones((1, m), f32), n2, jnp.zeros((_K - 5, m), f32)],
        axis=0)
    rowmin = None
    colmins = []
    for c in range(0, m, _MC):
        d = jnp.dot(aug1, aug2[:, c:c + _MC],
                    preferred_element_type=f32)
        rm = jnp.min(d, axis=1, keepdims=True)
        rowmin = rm if rowmin is None else jnp.minimum(rowmin, rm)
        colmins.append(jnp.min(d, axis=0, keepdims=True))
    rowmin = jnp.maximum(rowmin, 0.0)
    colmin = jnp.maximum(jnp.concatenate(colmins, axis=1), 0.0)
    s = (jnp.sum(rowmin, axis=(0, 1), keepdims=True) / (nbatch * n)
         + jnp.sum(colmin, axis=(0, 1), keepdims=True) / (nbatch * m))

    @pl.when(b == 0)
    def _():
        out_ref[...] = s

    @pl.when(b != 0)
    def _():
        out_ref[...] = out_ref[...] + s


def kernel(array1, array2):
    B, N, _ = array1.shape
    M = array2.shape[1]
    a2t = jnp.swapaxes(array2, 1, 2)
    out = pl.pallas_call(
        _chamfer_body,
        grid=(B,),
        in_specs=[
            pl.BlockSpec((1, N, 3), lambda b: (b, 0, 0)),
            pl.BlockSpec((1, 3, M), lambda b: (b, 0, 0)),
        ],
        out_specs=pl.BlockSpec((1, 1), lambda b: (0, 0)),
        out_shape=jax.ShapeDtypeStruct((1, 1), jnp.float32),
    )(array1, a2t)
    return out[0, 0]
